# Initial kernel scaffold; baseline (speedup 1.0000x reference)
import jax
import jax.numpy as jnp
from jax.experimental import pallas as pl


def kernel(x, wpe):
    raise NotImplementedError("write your pallas kernel here")


# speedup vs baseline: 5.0372x; 5.0372x over previous
import jax
import jax.numpy as jnp
from jax.experimental import pallas as pl


def _bcast_body(wpe_ref, out_ref):
    out_ref[...] = jnp.broadcast_to(wpe_ref[...][None], out_ref.shape)


def kernel(x, wpe):
    B, S = x.shape
    R, D = wpe.shape
    BLK = 512
    out = pl.pallas_call(
        _bcast_body,
        grid=(S // BLK,),
        in_specs=[pl.BlockSpec((BLK, D), lambda i: (i, 0))],
        out_specs=pl.BlockSpec((B, BLK, D), lambda i: (0, i, 0)),
        out_shape=jax.ShapeDtypeStruct((B, S, D), wpe.dtype),
    )(wpe)
    return out


# speedup vs baseline: 5.1854x; 1.0294x over previous
import jax
import jax.numpy as jnp
from jax.experimental import pallas as pl


def _bcast_body(wpe_ref, out_ref):
    out_ref[...] = jnp.broadcast_to(wpe_ref[...][None], out_ref.shape)


def kernel(x, wpe):
    B, S = x.shape
    R, D = wpe.shape
    BLK = 1024
    out = pl.pallas_call(
        _bcast_body,
        grid=(S // BLK,),
        in_specs=[pl.BlockSpec((BLK, D), lambda i: (i, 0))],
        out_specs=pl.BlockSpec((B, BLK, D), lambda i: (0, i, 0)),
        out_shape=jax.ShapeDtypeStruct((B, S, D), wpe.dtype),
    )(wpe)
    return out
